# baseline (device time: 79841 ns/iter reference)
import jax
import jax.numpy as jnp
from jax import lax
from jax.experimental import pallas as pl
from jax.experimental.pallas import tpu as pltpu

N_DEV = 4
CH = 256
NSUB = 4
SUB = CH // NSUB
HALF = N_DEV * CH
NHOP = 2 * (N_DEV - 1)


def _f(s):
    r = jnp.maximum(s, 0.0)
    return jnp.tanh(s) * s * s + r * r * r


def kernel(t):
    m, n = t.shape
    assert m == 2 * HALF

    def body(x_ref, out_ref, rs_f, rs_r, ssem, rsem):
        my = lax.axis_index("i")
        right = lax.rem(my + 1, N_DEV)
        left = lax.rem(my + 3, N_DEV)

        barrier_sem = pltpu.get_barrier_semaphore()
        for nbr in [left, right]:
            pl.semaphore_signal(
                barrier_sem, inc=1,
                device_id=(nbr,), device_id_type=pl.DeviceIdType.MESH,
            )
        pl.semaphore_wait(barrier_sem, 2)

        def rows(d, q, s):
            return pl.ds(d * HALF + q * CH + s * SUB, SUB)

        def rdma(src, dst, d, s, h, dev):
            return pltpu.make_async_remote_copy(
                src_ref=src, dst_ref=dst,
                send_sem=ssem.at[d, s, h], recv_sem=rsem.at[d, s, h],
                device_id=(dev,), device_id_type=pl.DeviceIdType.MESH,
            )

        dev_of = {0: right, 1: left}
        rs_of = {0: rs_f, 1: rs_r}
        def chunk_id(d, h):
            if h < N_DEV - 1:
                delta = -h if d == 0 else h
            else:
                ha = h - (N_DEV - 1)
                delta = 1 - ha if d == 0 else -1 + ha
            return lax.rem(my + delta + 2 * N_DEV, N_DEV)

        started = {}
        order = [(d, s) for s in range(NSUB) for d in (0, 1)]

        for h in range(N_DEV - 1):
            for d, s in order:
                c = chunk_id(d, h)
                if h == 0:
                    src = x_ref.at[rows(d, c, s)]
                else:
                    started[(d, s, h - 1)].wait_recv()
                    buf = rs_of[d]
                    sub = pl.ds(s * SUB, SUB)
                    buf[h - 1, sub, :] = buf[h - 1, sub, :] + x_ref[rows(d, c, s), :]
                    src = buf.at[h - 1, sub]
                r = rdma(src, rs_of[d].at[h, pl.ds(s * SUB, SUB)], d, s, h, dev_of[d])
                r.start()
                started[(d, s, h)] = r

        for d, s in order:
            started[(d, s, N_DEV - 2)].wait_recv()
            c = chunk_id(d, N_DEV - 1)
            sub = pl.ds(s * SUB, SUB)
            out_ref[rows(d, c, s), :] = _f(
                rs_of[d][N_DEV - 2, sub, :] + x_ref[rows(d, c, s), :]
            )
            r = rdma(out_ref.at[rows(d, c, s)], out_ref.at[rows(d, c, s)],
                     d, s, N_DEV - 1, dev_of[d])
            r.start()
            started[(d, s, N_DEV - 1)] = r

        for h in range(N_DEV, NHOP):
            for d, s in order:
                started[(d, s, h - 1)].wait_recv()
                c = chunk_id(d, h)
                r = rdma(out_ref.at[rows(d, c, s)], out_ref.at[rows(d, c, s)],
                         d, s, h, dev_of[d])
                r.start()
                started[(d, s, h)] = r

        for d, s in order:
            started[(d, s, NHOP - 1)].wait_recv()
        for r in started.values():
            r.wait_send()

    return pl.pallas_call(
        body,
        out_shape=jax.ShapeDtypeStruct((m, n), jnp.float32),
        in_specs=[pl.BlockSpec(memory_space=pltpu.VMEM)],
        out_specs=pl.BlockSpec(memory_space=pltpu.VMEM),
        scratch_shapes=[
            pltpu.VMEM((N_DEV - 1, CH, n), jnp.float32),
            pltpu.VMEM((N_DEV - 1, CH, n), jnp.float32),
            pltpu.SemaphoreType.DMA((2, NSUB, NHOP)),
            pltpu.SemaphoreType.DMA((2, NSUB, NHOP)),
        ],
        compiler_params=pltpu.CompilerParams(collective_id=0),
    )(t)
